# baseline (speedup 1.0000x reference)
import functools

import jax
import jax.numpy as jnp
from jax import lax
from jax.experimental import pallas as pl
from jax.experimental.pallas import tpu as pltpu
from jax.experimental.pallas import tpu_sc as plsc

BATCH = 16384
DIM = 16
NC = 2
NS = 16
NW = NC * NS
BPW = BATCH // NW
NB = 8
RB = BATCH // NB

_mesh = plsc.VectorSubcoreMesh(core_axis_name="c", subcore_axis_name="s")


@functools.partial(
    pl.kernel,
    mesh=_mesh,
    out_type=[
        jax.ShapeDtypeStruct((BATCH, DIM), jnp.float32),
        jax.ShapeDtypeStruct((BATCH, DIM), jnp.float32),
        jax.ShapeDtypeStruct((BATCH, DIM), jnp.float32),
    ],
    compiler_params=pltpu.CompilerParams(
        needs_layout_passes=False,
        use_tc_tiling_on_sc=False,
    ),
    scratch_types=[
        pltpu.VMEM((BPW,), jnp.int32),
        pltpu.VMEM((BPW,), jnp.int32),
        pltpu.VMEM((BPW,), jnp.int32),
        pltpu.VMEM((BPW, DIM), jnp.float32),
        pltpu.VMEM((BPW, DIM), jnp.float32),
        pltpu.VMEM((BPW, DIM), jnp.float32),
        pltpu.SemaphoreType.DMA,
    ],
)
def _sc_gather(users_hbm, pos_hbm, neg_hbm, ut_hbm, it_hbm,
               out_u, out_p, out_n, iu, ip, ineg, ru, rp, rn, sem):
    wid = lax.axis_index("s") * NC + lax.axis_index("c")
    base = wid * BPW

    pltpu.sync_copy(users_hbm.at[pl.ds(base, BPW)], iu)
    pltpu.sync_copy(pos_hbm.at[pl.ds(base, BPW)], ip)
    pltpu.sync_copy(neg_hbm.at[pl.ds(base, BPW)], ineg)

    c1 = pltpu.async_copy(ut_hbm.at[iu], ru, sem)
    c2 = pltpu.async_copy(it_hbm.at[ip], rp, sem)
    c3 = pltpu.async_copy(it_hbm.at[ineg], rn, sem)
    c1.wait()
    c2.wait()
    c3.wait()

    pltpu.sync_copy(ru, out_u.at[pl.ds(base, BPW)])
    pltpu.sync_copy(rp, out_p.at[pl.ds(base, BPW)])
    pltpu.sync_copy(rn, out_n.at[pl.ds(base, BPW)])


def _tc_loss_body(u_ref, p_ref, n_ref, o_ref):
    i = pl.program_id(0)
    u = u_ref[...]
    sp = jnp.sum(u * p_ref[...], axis=1)
    sn = jnp.sum(u * n_ref[...], axis=1)
    part = jnp.sum(1.0 / (1.0 + jnp.exp(-sp))) + jnp.sum(1.0 / (1.0 + jnp.exp(sn)))

    @pl.when(i == 0)
    def _init():
        o_ref[...] = jnp.zeros_like(o_ref)

    o_ref[...] += (-part / BATCH).reshape(1, 1)


_tc_loss = pl.pallas_call(
    _tc_loss_body,
    grid=(NB,),
    in_specs=[pl.BlockSpec((RB, DIM), lambda i: (i, 0)) for _ in range(3)],
    out_specs=pl.BlockSpec((1, 1), lambda i: (0, 0)),
    out_shape=jax.ShapeDtypeStruct((1, 1), jnp.float32),
)


def kernel(users, pos_items, neg_items, user_table, item_table):
    u = users.astype(jnp.int32)
    p = pos_items.astype(jnp.int32)
    n = neg_items.reshape(-1).astype(jnp.int32)
    gu, gp, gn = _sc_gather(u, p, n, user_table, item_table)
    loss = _tc_loss(gu, gp, gn)[0, 0]
    return (loss, loss, jnp.float32(0.0))


# speedup vs baseline: 6.0939x; 6.0939x over previous
import functools

import jax
import jax.numpy as jnp
from jax import lax
from jax.experimental import pallas as pl
from jax.experimental.pallas import tpu as pltpu
from jax.experimental.pallas import tpu_sc as plsc

N_ROWS = 1000000
BATCH = 16384
DIM = 16
NC = 2
NS = 16
NW = NC * NS
BPW = BATCH // NW
NSLICE = BPW // 16
RCH = 32768
CCH = -(-N_ROWS // RCH)


def _tc_soa_body(ut_ref, it_ref, *o_refs):
    for d in range(DIM):
        o_refs[d][...] = ut_ref[d, :]
        o_refs[DIM + d][...] = it_ref[d, :]


_tc_soa = pl.pallas_call(
    _tc_soa_body,
    grid=(CCH,),
    in_specs=[pl.BlockSpec((DIM, RCH), lambda c: (0, c)) for _ in range(2)],
    out_specs=[pl.BlockSpec((RCH,), lambda c: (c,)) for _ in range(2 * DIM)],
    out_shape=[jax.ShapeDtypeStruct((N_ROWS,), jnp.float32)
               for _ in range(2 * DIM)],
)

_mesh = plsc.VectorSubcoreMesh(core_axis_name="c", subcore_axis_name="s")


@functools.partial(
    pl.kernel,
    mesh=_mesh,
    out_type=jax.ShapeDtypeStruct((NW, 16), jnp.float32),
    compiler_params=pltpu.CompilerParams(
        needs_layout_passes=False,
        use_tc_tiling_on_sc=False,
    ),
    scratch_types=[
        pltpu.VMEM((BPW,), jnp.int32),
        pltpu.VMEM((BPW,), jnp.int32),
        pltpu.VMEM((BPW,), jnp.int32),
        pltpu.VMEM((DIM, BPW), jnp.float32),
        pltpu.VMEM((DIM, BPW), jnp.float32),
        pltpu.VMEM((DIM, BPW), jnp.float32),
        pltpu.VMEM((16,), jnp.float32),
        pltpu.SemaphoreType.DMA,
        pltpu.SemaphoreType.DMA,
        pltpu.SemaphoreType.DMA,
        pltpu.SemaphoreType.DMA,
    ],
)
def _sc_score(users_hbm, pos_hbm, neg_hbm, *rest):
    ut_planes = rest[:DIM]
    it_planes = rest[DIM:2 * DIM]
    out_hbm = rest[2 * DIM]
    iu, ip, ineg, ru, rp, rn, accv, si, su, sp_sem, sn_sem = rest[2 * DIM + 1:]

    wid = lax.axis_index("s") * NC + lax.axis_index("c")
    base = wid * BPW

    idx_copies = [
        pltpu.async_copy(users_hbm.at[pl.ds(base, BPW)], iu, si),
        pltpu.async_copy(pos_hbm.at[pl.ds(base, BPW)], ip, si),
        pltpu.async_copy(neg_hbm.at[pl.ds(base, BPW)], ineg, si),
    ]
    for c in idx_copies:
        c.wait()

    copies = []
    for d in range(DIM):
        copies.append(pltpu.async_copy(ut_planes[d].at[iu], ru.at[d], su))
        copies.append(pltpu.async_copy(it_planes[d].at[ip], rp.at[d], sp_sem))
        copies.append(pltpu.async_copy(it_planes[d].at[ineg], rn.at[d], sn_sem))
    for c in copies:
        c.wait()

    zero = jnp.zeros((16,), jnp.float32)

    def slice_step(s, acc):
        col = pl.ds(s * 16, 16)
        sp = zero
        sn = zero
        for d in range(DIM):
            uc = ru[d, col]
            sp = sp + uc * rp[d, col]
            sn = sn + uc * rn[d, col]
        acc = acc + 1.0 / (1.0 + jnp.exp(-sp)) + 1.0 / (1.0 + jnp.exp(sn))
        return acc

    acc = lax.fori_loop(0, NSLICE, slice_step, zero)
    accv[...] = acc
    pltpu.sync_copy(accv, out_hbm.at[wid])


def _tc_reduce_body(p_ref, o_ref):
    o_ref[...] = (-jnp.sum(p_ref[...]) / BATCH).reshape(1, 1)


_tc_reduce = pl.pallas_call(
    _tc_reduce_body,
    out_shape=jax.ShapeDtypeStruct((1, 1), jnp.float32),
)


def kernel(users, pos_items, neg_items, user_table, item_table):
    u = users.astype(jnp.int32)
    p = pos_items.astype(jnp.int32)
    n = neg_items.reshape(-1).astype(jnp.int32)
    planes = _tc_soa(user_table.T, item_table.T)
    partials = _sc_score(u, p, n, *planes)
    loss = _tc_reduce(partials)[0, 0]
    return (loss, loss, jnp.float32(0.0))
